# baseline (speedup 1.0000x reference)
import jax
import jax.numpy as jnp
from jax.experimental import pallas as pl
from jax.experimental.pallas import tpu as pltpu

C, P, F, B = 100, 64, 128, 4096
ALPHA = 5.0
EPS = 1e-08

TB = 2048
NBT = B // TB
CCHUNK = 20
RCHUNK = CCHUNK * P
NC = C // CCHUNK


def _cce_kernel(x_ref, tgt_ref, y_hbm, out_ref, ybuf, sems, acc_ref):
    i = pl.program_id(0)

    @pl.when(i == 0)
    def _start_copies():
        for j in range(NC):
            pltpu.make_async_copy(y_hbm.at[j], ybuf.at[j], sems.at[j]).start()

    x = x_ref[...]
    x2 = jnp.sum(x * x, axis=1)
    xm = -2.0 * x
    tgt = tgt_ref[0, 0, :]

    tmin = jnp.full((TB,), jnp.inf, jnp.float32)
    wmin = jnp.full((TB,), jnp.inf, jnp.float32)

    for j in range(NC):
        @pl.when(i == 0)
        def _wait(j=j):
            pltpu.make_async_copy(y_hbm.at[j], ybuf.at[j], sems.at[j]).wait()

        y = ybuf[j]
        y2 = jnp.sum(y * y, axis=1)
        s = y2[:, None] + jax.lax.dot_general(
            y, xm, (((1,), (1,)), ((), ())),
            preferred_element_type=jnp.float32)
        m = jnp.min(s.reshape(CCHUNK, P, TB), axis=1)
        cls = jax.lax.broadcasted_iota(jnp.int32, (CCHUNK, TB), 0) + j * CCHUNK
        is_t = cls == tgt[None, :]
        tmin = jnp.minimum(tmin, jnp.min(jnp.where(is_t, m, jnp.inf), axis=0))
        wmin = jnp.minimum(wmin, jnp.min(jnp.where(is_t, jnp.inf, m), axis=0))

    t = jnp.maximum(tmin + x2, 0.0)
    w = jnp.maximum(wmin + x2, 0.0)
    tp = jnp.sum(t.reshape(TB // 128, 128), axis=0)
    wp = jnp.sum(w.reshape(TB // 128, 128), axis=0)

    @pl.when(i == 0)
    def _init():
        acc_ref[...] = jnp.zeros_like(acc_ref)

    acc_ref[...] += jnp.stack([tp, wp])

    @pl.when(i == NBT - 1)
    def _finish():
        denom = float(B * F)
        target_loss = jnp.sum(acc_ref[0, :]) / denom
        non_target_loss = jnp.sum(acc_ref[1, :]) / denom
        out_ref[0] = (1.0 - ALPHA) * target_loss \
            + ALPHA / (non_target_loss + EPS)


@jax.jit
def kernel(outputs, target_classes, clusters):
    y = clusters.reshape(NC, RCHUNK, F)
    tgt = target_classes.astype(jnp.int32).reshape(NBT, 1, TB)

    loss = pl.pallas_call(
        _cce_kernel,
        grid=(NBT,),
        in_specs=[
            pl.BlockSpec((TB, F), lambda i: (i, 0)),
            pl.BlockSpec((1, 1, TB), lambda i: (i, 0, 0)),
            pl.BlockSpec(memory_space=pltpu.MemorySpace.HBM),
        ],
        out_specs=pl.BlockSpec(memory_space=pltpu.SMEM),
        out_shape=jax.ShapeDtypeStruct((1,), jnp.float32),
        scratch_shapes=[
            pltpu.VMEM((NC, RCHUNK, F), jnp.float32),
            pltpu.SemaphoreType.DMA((NC,)),
            pltpu.VMEM((2, 128), jnp.float32),
        ],
    )(outputs, tgt, y)

    return loss[0]
